# Initial kernel scaffold; baseline (speedup 1.0000x reference)
import jax
import jax.numpy as jnp
from jax.experimental import pallas as pl


def kernel(donors_idx, mask_fit_X_col):
    raise NotImplementedError("write your pallas kernel here")


# speedup vs baseline: 1.4354x; 1.4354x over previous
import functools

import jax
import jax.numpy as jnp
from jax import lax
from jax.experimental import pallas as pl
from jax.experimental.pallas import tpu as pltpu
from jax.experimental.pallas import tpu_sc as plsc

_NC, _NS, _L = 2, 16, 16
_NW = _NC * _NS

_B = 16384 * 100
_BPW = _B // _NW

_mesh = plsc.VectorSubcoreMesh(core_axis_name="c", subcore_axis_name="s")


@functools.partial(
    pl.kernel,
    out_type=jax.ShapeDtypeStruct((_B,), jnp.float32),
    mesh=_mesh,
    scratch_types=[
        pltpu.VMEM((_BPW,), jnp.int32),
        pltpu.VMEM((_BPW,), jnp.float32),
        pltpu.SemaphoreType.DMA,
    ],
)
def _gather_mask(idx_hbm, table_hbm, out_hbm, idx_v, vals_v, sem):
    wid = lax.axis_index("s") * _NC + lax.axis_index("c")
    base = wid * _BPW
    pltpu.sync_copy(idx_hbm.at[pl.ds(base, _BPW)], idx_v)
    pltpu.async_copy(table_hbm.at[idx_v], vals_v, sem).wait()

    @pl.loop(jnp.int32(0), jnp.int32(_BPW), step=jnp.int32(_L))
    def _(off):
        sl = pl.ds(off, _L)
        vals_v[sl] = 1.0 - vals_v[sl]
    pltpu.sync_copy(vals_v, out_hbm.at[pl.ds(base, _BPW)])


def kernel(donors_idx, mask_fit_X_col):
    idx32 = donors_idx.astype(jnp.int32).reshape(-1)
    masked = _gather_mask(idx32, mask_fit_X_col)
    return masked.reshape(donors_idx.shape).astype(donors_idx.dtype)
